# baseline (speedup 1.0000x reference)
import functools

import jax
import jax.numpy as jnp
from jax import lax
from jax.experimental import pallas as pl
from jax.experimental.pallas import tpu as pltpu
from jax.experimental.pallas import tpu_sc as plsc

N = 10000
E = 320000
D = 128
NC = 2
NS = 16
DH = D // NC
NP2 = N // 2
CH = 128
NCH = 160
EP = NCH * CH
NB = 2
ZR = 40
NZC = NP2 // ZR

_mesh = plsc.VectorSubcoreMesh(core_axis_name="c", subcore_axis_name="s")


def _lane_bcast(v16, j):
    return lax.gather(
        v16, jnp.full((16, 1), j, jnp.int32),
        dimension_numbers=lax.GatherDimensionNumbers(
            offset_dims=(), collapsed_slice_dims=(0,), start_index_map=(0,)),
        slice_sizes=(1,),
        mode=lax.GatherScatterMode.PROMISE_IN_BOUNDS)


_SCRATCH = (
    [pltpu.VMEM((CH, D), jnp.float32)] * NB
    + [pltpu.VMEM((CH,), jnp.int32)] * NB
    + [pltpu.VMEM((CH,), jnp.int32)] * NB
    + [pltpu.VMEM((CH,), jnp.int32)] * NB
    + [pltpu.VMEM((CH,), jnp.int32)] * NB
    + [pltpu.VMEM((CH,), jnp.float32)] * NB
    + [pltpu.SemaphoreType.DMA] * (5 * NB)
    + [pltpu.VMEM((CH * 64,), jnp.float32)]
    + [pltpu.VMEM_SHARED((NP2, D), jnp.float32)]
    + [pltpu.VMEM_SHARED((NP2, D), jnp.float32)]
)


@functools.partial(
    pl.kernel,
    out_type=jax.ShapeDtypeStruct((NC, NP2, D), jnp.float32),
    mesh=_mesh,
    scratch_types=_SCRATCH,
)
def _propagate(x_hbm, src_hbm, dst_hbm, w_hbm, out_hbm, *sc):
    rows = sc[0:NB]
    srcb = sc[NB:2 * NB]
    dstb = sc[2 * NB:3 * NB]
    srcb2 = sc[3 * NB:4 * NB]
    dstb2 = sc[4 * NB:5 * NB]
    wb = sc[5 * NB:6 * NB]
    gs = sc[6 * NB:7 * NB]
    ss = sc[7 * NB:8 * NB]
    ls = sc[8 * NB:9 * NB]
    ld = sc[9 * NB:10 * NB]
    lw = sc[10 * NB:11 * NB]
    wexp = sc[11 * NB]
    xsp = sc[11 * NB + 1]
    acc_sh = sc[11 * NB + 2]

    cid = lax.axis_index("c")
    sid = lax.axis_index("s")

    zeros16 = jnp.zeros((16,), jnp.float32)
    ones16 = jnp.ones((16,), jnp.float32)
    my_chunks = jnp.where(sid < NZC % NS, NZC // NS + 1, NZC // NS)

    def zfill(i, carry):
        for cc in range(D // 16):
            rows[0][i, pl.ds(cc * 16, 16)] = zeros16
        return carry

    lax.fori_loop(0, ZR, zfill, 0)

    def zcopy(k, carry):
        r0 = pl.multiple_of((sid + k * NS) * ZR, 8)
        xr0 = pl.multiple_of(cid * NP2 + r0, 8)
        pltpu.sync_copy(x_hbm.at[pl.ds(xr0, ZR)], xsp.at[pl.ds(r0, ZR)])
        pltpu.sync_copy(rows[0].at[pl.ds(0, ZR)], acc_sh.at[pl.ds(r0, ZR)])
        return carry

    lax.fori_loop(0, my_chunks, zcopy, 0)
    plsc.subcore_barrier()

    def eslice(c, hbm):
        return hbm.at[pl.ds(pl.multiple_of(sid * EP + c * CH, 8), CH)]

    def start_load(c, hbm, buf, sem):
        pltpu.async_copy(eslice(c, hbm), buf, sem)

    def wait_load(c, hbm, buf, sem):
        pltpu.make_async_copy(eslice(c, hbm), buf, sem).wait()

    def halve(src_ref, dst_ref):
        def hbody(g, carry):
            sl = pl.ds(g * 16, 16)
            dst_ref[sl] = lax.shift_right_logical(src_ref[sl], 1)
            return carry

        lax.fori_loop(0, CH // 16, hbody, 0)

    def start_gather(b):
        pltpu.async_copy(xsp.at[srcb2[b]], rows[b], gs[b])

    def wait_gather(b):
        pltpu.make_async_copy(xsp.at[srcb2[b]], rows[b], gs[b]).wait()

    def start_scatter(b):
        pltpu.async_copy(rows[b], acc_sh.at[dstb2[b]], ss[b], add=True)

    def wait_scatter(b):
        pltpu.make_async_copy(rows[b], acc_sh.at[dstb2[b]], ss[b]).wait()

    def scale(b):
        def wexp_body(g, gcarry):
            sl = pl.ds(g * 16, 16)
            w16 = wb[b][sl]
            sp = (srcb[b][sl] & 1).astype(jnp.float32)
            dp = (dstb[b][sl] & 1).astype(jnp.float32)
            osp = ones16 - sp
            odp = ones16 - dp
            cll = w16 * osp * odp
            clh = w16 * sp * odp
            chl = w16 * osp * dp
            chh = w16 * sp * dp
            for j in range(16):
                e64 = (g * 16 + j) * 64
                wexp[pl.ds(e64, 16)] = _lane_bcast(cll, j)
                wexp[pl.ds(e64 + 16, 16)] = _lane_bcast(clh, j)
                wexp[pl.ds(e64 + 32, 16)] = _lane_bcast(chl, j)
                wexp[pl.ds(e64 + 48, 16)] = _lane_bcast(chh, j)
            return gcarry

        lax.fori_loop(0, CH // 16, wexp_body, 0)

        def edge_body(eh, ecarry):
            for u in range(2):
                e = eh * 2 + u
                e64 = pl.multiple_of(e * 64, 8)
                cll = wexp[pl.ds(e64, 16)]
                clh = wexp[pl.ds(e64 + 16, 16)]
                chl = wexp[pl.ds(e64 + 32, 16)]
                chh = wexp[pl.ds(e64 + 48, 16)]
                for cc in range(DH // 16):
                    lo = pl.ds(cc * 16, 16)
                    hi = pl.ds(DH + cc * 16, 16)
                    a = rows[b][e, lo]
                    bv = rows[b][e, hi]
                    rows[b][e, lo] = a * cll + bv * clh
                    rows[b][e, hi] = a * chl + bv * chh
            return ecarry

        lax.fori_loop(0, CH // 2, edge_body, 0)

    start_load(0, src_hbm, srcb[0], ls[0])
    start_load(1, src_hbm, srcb[1], ls[1])
    start_load(0, dst_hbm, dstb[0], ld[0])
    start_load(0, w_hbm, wb[0], lw[0])
    start_load(1, w_hbm, wb[1], lw[1])
    wait_load(0, src_hbm, srcb[0], ls[0])
    halve(srcb[0], srcb2[0])
    start_gather(0)

    def pair_body(p, carry):
        c0 = 2 * p
        c1 = c0 + 1
        c2 = jnp.minimum(c0 + 2, NCH - 1)
        c3 = jnp.minimum(c0 + 3, NCH - 1)

        @pl.when(p > 0)
        def _():
            wait_scatter(1)
        start_load(c1, dst_hbm, dstb[1], ld[1])
        wait_load(c1, src_hbm, srcb[1], ls[1])
        halve(srcb[1], srcb2[1])
        start_gather(1)
        wait_gather(0)
        wait_load(c0, w_hbm, wb[0], lw[0])
        wait_load(c0, dst_hbm, dstb[0], ld[0])
        scale(0)
        halve(dstb[0], dstb2[0])
        start_load(c2, w_hbm, wb[0], lw[0])
        start_scatter(0)
        start_load(c2, src_hbm, srcb[0], ls[0])
        wait_gather(1)
        wait_load(c1, w_hbm, wb[1], lw[1])
        wait_load(c1, dst_hbm, dstb[1], ld[1])
        scale(1)
        halve(dstb[1], dstb2[1])
        start_load(c3, w_hbm, wb[1], lw[1])
        start_scatter(1)
        start_load(c3, src_hbm, srcb[1], ls[1])
        wait_scatter(0)
        start_load(c2, dst_hbm, dstb[0], ld[0])
        wait_load(c2, src_hbm, srcb[0], ls[0])
        halve(srcb[0], srcb2[0])
        start_gather(0)
        return carry

    lax.fori_loop(0, NCH // 2, pair_body, 0)

    last = NCH - 1
    wait_load(last, src_hbm, srcb[1], ls[1])
    wait_load(last, w_hbm, wb[0], lw[0])
    wait_load(last, w_hbm, wb[1], lw[1])
    wait_load(last, dst_hbm, dstb[0], ld[0])
    wait_gather(0)
    wait_scatter(1)
    plsc.subcore_barrier()

    def ocopy(k, carry):
        r0 = pl.multiple_of((sid + k * NS) * ZR, 8)
        pltpu.sync_copy(acc_sh.at[pl.ds(r0, ZR)],
                        out_hbm.at[cid, pl.ds(r0, ZR)])
        return carry

    lax.fori_loop(0, my_chunks, ocopy, 0)


_BM = 2000
_BMH = _BM // 2


def _mm_body(h0_ref, h1_ref, w_ref, o_ref):
    o_ref[...] = (
        jnp.dot(h0_ref[...], w_ref[0:DH, :],
                preferred_element_type=jnp.float32)
        + jnp.dot(h1_ref[...], w_ref[DH:D, :],
                  preferred_element_type=jnp.float32))


def _matmul(hp, W0):
    h0 = hp[0].reshape(N, DH)
    h1 = hp[1].reshape(N, DH)
    return pl.pallas_call(
        _mm_body,
        grid=(N // _BM,),
        in_specs=[
            pl.BlockSpec((_BM, DH), lambda i: (i, 0)),
            pl.BlockSpec((_BM, DH), lambda i: (i, 0)),
            pl.BlockSpec((D, D), lambda i: (0, 0)),
        ],
        out_specs=pl.BlockSpec((_BM, D), lambda i: (i, 0)),
        out_shape=jax.ShapeDtypeStruct((N, D), jnp.float32),
    )(h0, h1, W0)


def kernel(x, edge_index, edge_weight, W0):
    pad = NS * EP - E
    zi = jnp.zeros((pad,), jnp.int32)
    dst = jnp.concatenate([edge_index[0].astype(jnp.int32), zi])
    src = jnp.concatenate([edge_index[1].astype(jnp.int32), zi])
    w = jnp.concatenate([edge_weight.astype(jnp.float32),
                         jnp.zeros((pad,), jnp.float32)])
    xpk = (x.reshape(N, NC, DH).transpose(1, 0, 2)
           .reshape(NC * NP2, D))
    hp = _propagate(xpk, src, dst, w)
    return _matmul(hp, W0)


# speedup vs baseline: 1.1147x; 1.1147x over previous
import functools

import jax
import jax.numpy as jnp
from jax import lax
from jax.experimental import pallas as pl
from jax.experimental.pallas import tpu as pltpu
from jax.experimental.pallas import tpu_sc as plsc

N = 10000
E = 320000
D = 128
NC = 2
NS = 16
NW = NC * NS
EP = E // NW
CH = 80
NCH = EP // CH
ZR = 40
NZC = N // ZR

_mesh = plsc.VectorSubcoreMesh(core_axis_name="c", subcore_axis_name="s")


def _lane_bcast(v16, j):
    return lax.gather(
        v16, jnp.full((16, 1), j, jnp.int32),
        dimension_numbers=lax.GatherDimensionNumbers(
            offset_dims=(), collapsed_slice_dims=(0,), start_index_map=(0,)),
        slice_sizes=(1,),
        mode=lax.GatherScatterMode.PROMISE_IN_BOUNDS)


@functools.partial(
    pl.kernel,
    out_type=jax.ShapeDtypeStruct((NC, N, D), jnp.float32),
    mesh=_mesh,
    scratch_types=[
        pltpu.VMEM((CH,), jnp.int32),
        pltpu.VMEM((CH,), jnp.int32),
        pltpu.VMEM((CH,), jnp.float32),
        pltpu.VMEM((CH, D), jnp.float32),
        pltpu.VMEM((ZR, D), jnp.float32),
        pltpu.VMEM_SHARED((N, D), jnp.float32),
        pltpu.SemaphoreType.DMA,
    ],
)
def _propagate(x_hbm, src_hbm, dst_hbm, w_hbm, out_hbm,
               src_v, dst_v, w_v, rows_v, zero_v, acc_sh, sem):
    cid = lax.axis_index("c")
    sid = lax.axis_index("s")
    wid = cid * NS + sid

    zeros16 = jnp.zeros((16,), jnp.float32)
    my_chunks = jnp.where(sid < NZC % NS, NZC // NS + 1, NZC // NS)

    def zfill(i, carry):
        for cc in range(D // 16):
            zero_v[i, pl.ds(cc * 16, 16)] = zeros16
        return carry

    lax.fori_loop(0, ZR, zfill, 0)

    def zcopy(k, carry):
        r0 = pl.multiple_of((sid + k * NS) * ZR, 8)
        pltpu.sync_copy(zero_v, acc_sh.at[pl.ds(r0, ZR)])
        return carry

    lax.fori_loop(0, my_chunks, zcopy, 0)
    plsc.subcore_barrier()

    def chunk_body(c, carry):
        base = pl.multiple_of(wid * EP + c * CH, 8)
        pltpu.sync_copy(src_hbm.at[pl.ds(base, CH)], src_v)
        pltpu.sync_copy(dst_hbm.at[pl.ds(base, CH)], dst_v)
        pltpu.sync_copy(w_hbm.at[pl.ds(base, CH)], w_v)
        pltpu.async_copy(x_hbm.at[src_v], rows_v, sem).wait()

        def group_body(g, gcarry):
            w16 = w_v[pl.ds(g * 16, 16)]
            for j in range(16):
                wspl = _lane_bcast(w16, j)
                e = g * 16 + j
                for cc in range(D // 16):
                    sl = pl.ds(cc * 16, 16)
                    rows_v[e, sl] = rows_v[e, sl] * wspl
            return gcarry

        lax.fori_loop(0, CH // 16, group_body, 0)
        pltpu.sync_copy(rows_v, acc_sh.at[dst_v], add=True)
        return carry

    lax.fori_loop(0, NCH, chunk_body, 0)
    plsc.subcore_barrier()

    def ocopy(k, carry):
        r0 = pl.multiple_of((sid + k * NS) * ZR, 8)
        pltpu.sync_copy(acc_sh.at[pl.ds(r0, ZR)],
                        out_hbm.at[cid, pl.ds(r0, ZR)])
        return carry

    lax.fori_loop(0, my_chunks, ocopy, 0)


_BM = 2000


def _mm_body(hp_ref, w_ref, o_ref):
    h = hp_ref[0] + hp_ref[1]
    o_ref[...] = jnp.dot(h, w_ref[...], preferred_element_type=jnp.float32)


def _matmul(hp, W0):
    return pl.pallas_call(
        _mm_body,
        grid=(N // _BM,),
        in_specs=[
            pl.BlockSpec((NC, _BM, D), lambda i: (0, i, 0)),
            pl.BlockSpec((D, D), lambda i: (0, 0)),
        ],
        out_specs=pl.BlockSpec((_BM, D), lambda i: (i, 0)),
        out_shape=jax.ShapeDtypeStruct((N, D), jnp.float32),
    )(hp, W0)


def kernel(x, edge_index, edge_weight, W0):
    dst = edge_index[0].astype(jnp.int32)
    src = edge_index[1].astype(jnp.int32)
    hp = _propagate(x, src, dst, edge_weight.astype(jnp.float32))
    return _matmul(hp, W0)
